# baseline (speedup 1.0000x reference)
import functools

import jax
import jax.numpy as jnp
from jax import lax
from jax.experimental import pallas as pl
from jax.experimental.pallas import tpu as pltpu
from jax.experimental.pallas import tpu_sc as plsc

MAX_LEN = 8192
HIDDEN = 1024
BATCH = 16384

NUM_CORES = 2
NUM_SUBCORES = 16
NUM_WORKERS = NUM_CORES * NUM_SUBCORES
B_PER_W = BATCH // NUM_WORKERS
CHUNK = 16
NUM_CHUNKS = B_PER_W // CHUNK

_mesh = plsc.VectorSubcoreMesh(core_axis_name="c", subcore_axis_name="s")


@functools.partial(
    pl.kernel,
    mesh=_mesh,
    out_type=jax.ShapeDtypeStruct((BATCH, HIDDEN), jnp.float32),
    scratch_types=[
        pltpu.VMEM((NUM_CHUNKS, CHUNK), jnp.int32),
        *([pltpu.VMEM((CHUNK, HIDDEN), jnp.float32)] * 7),
        *([pltpu.SemaphoreType.DMA] * 14),
    ],
)
def _pe_gather(idx_hbm, table_hbm, out_hbm, idx_v, *scratch):
    wid = lax.axis_index("s") * NUM_CORES + lax.axis_index("c")
    base = wid * B_PER_W
    pltpu.sync_copy(idx_hbm.at[wid], idx_v)

    NBUF = 7
    bufs = scratch[:NBUF]
    gsems = scratch[NBUF:2 * NBUF]
    ssems = scratch[2 * NBUF:]
    gather = [None] * NBUF
    scatter = [None] * NBUF

    for j in range(min(NBUF - 1, NUM_CHUNKS)):
        gather[j] = pltpu.async_copy(table_hbm.at[idx_v.at[j]], bufs[j], gsems[j])
    for j in range(NUM_CHUNKS):
        b = j % NBUF
        nxt = j + NBUF - 1
        if nxt < NUM_CHUNKS:
            nb = nxt % NBUF
            if scatter[nb] is not None:
                scatter[nb].wait()
            gather[nb] = pltpu.async_copy(
                table_hbm.at[idx_v.at[nxt]], bufs[nb], gsems[nb]
            )
        gather[b].wait()
        scatter[b] = pltpu.async_copy(
            bufs[b], out_hbm.at[pl.ds(base + j * CHUNK, CHUNK)], ssems[b]
        )
    for b in range(NBUF):
        if scatter[b] is not None:
            scatter[b].wait()


def kernel(timestep, pe):
    idx = timestep.astype(jnp.int32).reshape(NUM_WORKERS, NUM_CHUNKS, CHUNK)
    return _pe_gather(idx, pe)


# speedup vs baseline: 1.0115x; 1.0115x over previous
import functools

import jax
import jax.numpy as jnp
from jax import lax
from jax.experimental import pallas as pl
from jax.experimental.pallas import tpu as pltpu
from jax.experimental.pallas import tpu_sc as plsc

MAX_LEN = 8192
HIDDEN = 1024
BATCH = 16384

NUM_CORES = 2
NUM_SUBCORES = 16
NUM_WORKERS = NUM_CORES * NUM_SUBCORES
B_PER_W = BATCH // NUM_WORKERS
CHUNK = 16
NUM_CHUNKS = B_PER_W // CHUNK

_mesh = plsc.VectorSubcoreMesh(core_axis_name="c", subcore_axis_name="s")


@functools.partial(
    pl.kernel,
    mesh=_mesh,
    out_type=jax.ShapeDtypeStruct((BATCH, HIDDEN), jnp.float32),
    scratch_types=[
        pltpu.VMEM((NUM_CHUNKS, CHUNK), jnp.int32),
        *([pltpu.VMEM((CHUNK, HIDDEN), jnp.float32)] * 6),
        *([pltpu.SemaphoreType.DMA] * 12),
    ],
)
def _pe_gather(idx_hbm, table_hbm, out_hbm, idx_v, *scratch):
    wid = lax.axis_index("s") * NUM_CORES + lax.axis_index("c")
    base = wid * B_PER_W
    pltpu.sync_copy(idx_hbm.at[wid], idx_v)

    NBUF = 6
    bufs = scratch[:NBUF]
    gsems = scratch[NBUF:2 * NBUF]
    ssems = scratch[2 * NBUF:]
    gather = [None] * NBUF
    scatter = [None] * NBUF

    for j in range(min(NBUF - 1, NUM_CHUNKS)):
        gather[j] = pltpu.async_copy(table_hbm.at[idx_v.at[j]], bufs[j], gsems[j])
    for j in range(NUM_CHUNKS):
        b = j % NBUF
        nxt = j + NBUF - 1
        if nxt < NUM_CHUNKS:
            nb = nxt % NBUF
            if scatter[nb] is not None:
                scatter[nb].wait()
            gather[nb] = pltpu.async_copy(
                table_hbm.at[idx_v.at[nxt]], bufs[nb], gsems[nb]
            )
        gather[b].wait()
        scatter[b] = pltpu.async_copy(
            bufs[b], out_hbm.at[pl.ds(base + j * CHUNK, CHUNK)], ssems[b]
        )
    for b in range(NBUF):
        if scatter[b] is not None:
            scatter[b].wait()


def kernel(timestep, pe):
    idx = timestep.astype(jnp.int32).reshape(NUM_WORKERS, NUM_CHUNKS, CHUNK)
    return _pe_gather(idx, pe)
